# baseline (speedup 1.0000x reference)
import functools

import jax
import jax.numpy as jnp
from jax import lax
from jax.experimental import pallas as pl
from jax.experimental.pallas import tpu as pltpu
from jax.experimental.pallas import tpu_sc as plsc

SEQ_LEN = 4096
D = 768
NUM_FRAMES = 81
NUM_WINDOWS = 21
WIN = 203
NC, NS = 2, 16
NW = NC * NS


def _window_starts():
    tokens_per_frame = SEQ_LEN / NUM_FRAMES
    half = int(tokens_per_frame * 4 / 2)
    pos = []
    for i in range(NUM_WINDOWS):
        if i == 0:
            pos.append(0)
        else:
            st = tokens_per_frame * ((i - 1) * 4 + 1)
            en = tokens_per_frame * (i * 4 + 1)
            pos.append(int((st + en) / 2) - 1)
    ranges = [[p - half, p + half] for p in pos]
    ranges[0] = [-(half * 2 - ranges[1][0]), ranges[1][0]]
    return ranges


_RANGES = _window_starts()
_K_LENS = []
for _s, _e in _RANGES:
    _vs, _ve = max(_s, 0), min(_e, SEQ_LEN - 1)
    _K_LENS.append(_ve - _vs + 1 if _vs <= _ve else 0)

_COPY_SEGS = []
for _w, (_s, _e) in enumerate(_RANGES):
    _vs, _ve = max(_s, 0), min(_e, SEQ_LEN - 1)
    _COPY_SEGS.append((_w * WIN, _vs, _ve - _vs + 1))
_TOTAL_COPY = sum(n for _, _, n in _COPY_SEGS)
_ZERO_START, _ZERO_N = _K_LENS[0], WIN - _K_LENS[0]

_CHUNK = _TOTAL_COPY // NW
_WORKER_SEGS = []
_c_bounds = []
_acc = 0
for _, _, _n in _COPY_SEGS:
    _c_bounds.append((_acc, _acc + _n))
    _acc += _n
for _wk in range(NW):
    _c0 = _wk * _CHUNK
    _c1 = _TOTAL_COPY if _wk == NW - 1 else (_wk + 1) * _CHUNK
    _segs = []
    for (_dst, _src, _n), (_b0, _b1) in zip(_COPY_SEGS, _c_bounds):
        _lo, _hi = max(_c0, _b0), min(_c1, _b1)
        if _lo < _hi:
            _segs.append((_lo - _c0, _src + (_lo - _b0), _dst + (_lo - _b0), _hi - _lo))
    _WORKER_SEGS.append(_segs)
_MAX_ROWS = max(sum(s[3] for s in segs) for segs in _WORKER_SEGS)

_ZPER = -(-_ZERO_N // (NW - 1))
_WORKER_ZERO = []
_zoff = 0
for _wk in range(NW):
    _n = min(_ZPER, _ZERO_N - _zoff)
    if _n > 0:
        _WORKER_ZERO.append((_ZERO_START + _zoff, _n))
        _zoff += _n
    else:
        _WORKER_ZERO.append(None)

_TOTAL_ROWS = NUM_WINDOWS * WIN


def _sc_body(src_hbm, out_hbm, buf, zbuf):
    wid = lax.axis_index("s") * NC + lax.axis_index("c")

    z16 = jnp.zeros((16,), jnp.float32)
    for i in range(_ZPER * D // 16):
        zbuf[pl.ds(i * 16, 16)] = z16

    def _make(wk):
        segs = _WORKER_SEGS[wk]
        zfill = _WORKER_ZERO[wk]

        def _run():
            for bo, ss, _, ln in segs:
                pltpu.sync_copy(src_hbm.at[pl.ds(ss * D, ln * D)],
                                buf.at[pl.ds(bo * D, ln * D)])
            for bo, _, ds, ln in segs:
                pltpu.sync_copy(buf.at[pl.ds(bo * D, ln * D)],
                                out_hbm.at[pl.ds(ds * D, ln * D)])
            if zfill is not None:
                dz, nz = zfill
                pltpu.sync_copy(zbuf.at[pl.ds(0, nz * D)],
                                out_hbm.at[pl.ds(dz * D, nz * D)])

        return _run

    for wk in range(NW):
        pl.when(wid == wk)(_make(wk))


@functools.partial(jax.jit, static_argnames=())
def _sc_copy(src):
    mesh = plsc.VectorSubcoreMesh(core_axis_name="c", subcore_axis_name="s")
    flat = pl.kernel(
        _sc_body,
        out_type=jax.ShapeDtypeStruct((_TOTAL_ROWS * D,), jnp.float32),
        mesh=mesh,
        scratch_types=[
            pltpu.VMEM((_MAX_ROWS * D,), jnp.float32),
            pltpu.VMEM((_ZPER * D,), jnp.float32),
        ],
    )(src.reshape(-1))
    return flat


def kernel(audio_proj, num_frames):
    del num_frames
    flat = _sc_copy(audio_proj)
    sub_sequences = flat.reshape(1, NUM_WINDOWS, WIN, D)
    k_lens = jnp.asarray(_K_LENS, dtype=jnp.int32)
    return sub_sequences, k_lens


# speedup vs baseline: 1.0420x; 1.0420x over previous
import functools

import jax
import jax.numpy as jnp
from jax import lax
from jax.experimental import pallas as pl
from jax.experimental.pallas import tpu as pltpu
from jax.experimental.pallas import tpu_sc as plsc

SEQ_LEN = 4096
D = 768
NUM_WINDOWS = 21
WIN = 203
WIN_PAD = 208
NC, NS = 2, 16
NW = NC * NS
IDXN = 112


def _window_ranges():
    tokens_per_frame = SEQ_LEN / 81
    half = int(tokens_per_frame * 4 / 2)
    pos = []
    for i in range(NUM_WINDOWS):
        if i == 0:
            pos.append(0)
        else:
            st = tokens_per_frame * ((i - 1) * 4 + 1)
            en = tokens_per_frame * (i * 4 + 1)
            pos.append(int((st + en) / 2) - 1)
    ranges = [[p - half, p + half] for p in pos]
    ranges[0] = [-(half * 2 - ranges[1][0]), ranges[1][0]]
    return ranges


_RANGES = _window_ranges()
_K_LENS = []
for _s, _e in _RANGES:
    _vs, _ve = max(_s, 0), min(_e, SEQ_LEN - 1)
    _K_LENS.append(_ve - _vs + 1 if _vs <= _ve else 0)
_KLEN0 = _K_LENS[0]
_STARTS = [max(_s, 0) for _s, _ in _RANGES]


def _sc_body(src_hbm, zeros_hbm, out_hbm, idx_v, rows_v, sem):
    wid = lax.axis_index("s") * NC + lax.axis_index("c")
    iota16 = lax.iota(jnp.int32, 16)
    z16 = jnp.zeros((16,), jnp.float32)

    def _fill_idx(base, clamp):
        for k in range(IDXN // 16):
            vals = jnp.minimum(iota16 + (16 * k), clamp) + base
            idx_v[pl.ds(16 * k, 16)] = vals

    def _gather():
        pltpu.async_copy(src_hbm.at[idx_v], rows_v.at[pl.ds(0, IDXN)], sem).wait()

    def _unit_a(w):
        def _run():
            if w == 0:
                _fill_idx(0, _KLEN0 - 1)
                _gather()
                for r in range(_KLEN0, 56):
                    for cb in range(D // 16):
                        rows_v[r, pl.ds(16 * cb, 16)] = z16
                pltpu.sync_copy(rows_v.at[pl.ds(0, 48)],
                                out_hbm.at[0, pl.ds(0, 48), :])
                pltpu.sync_copy(rows_v.at[pl.ds(48, 8)],
                                out_hbm.at[0, pl.ds(48, 8), :])
            else:
                _fill_idx(_STARTS[w], WIN - 1)
                _gather()
                pltpu.sync_copy(rows_v.at[pl.ds(0, 104)],
                                out_hbm.at[w, pl.ds(0, 104), :])
        return _run

    def _unit_b(w):
        def _run():
            if w == 0:
                pltpu.sync_copy(zeros_hbm, rows_v.at[pl.ds(0, 104)])
            else:
                _fill_idx(_STARTS[w] + 104, WIN - 1 - 104)
                _gather()
            pltpu.sync_copy(rows_v.at[pl.ds(0, 104)],
                            out_hbm.at[w, pl.ds(104, 104), :])
        return _run

    def _unit_z():
        def _run():
            pltpu.sync_copy(zeros_hbm.at[pl.ds(0, 48)], rows_v.at[pl.ds(0, 48)])
            pltpu.sync_copy(rows_v.at[pl.ds(0, 48)],
                            out_hbm.at[0, pl.ds(56, 48), :])
        return _run

    units = [_unit_a(w) for w in range(NUM_WINDOWS)]
    tail = [_unit_b(w) for w in range(NUM_WINDOWS)] + [_unit_z()]
    for wk in range(21):
        pl.when(wid == wk)(units[wk])
    for i in range(11):
        def _pair(a, b):
            def _run():
                a()
                b()
            return _run
        pl.when(wid == 21 + i)(_pair(tail[2 * i], tail[2 * i + 1]))


@functools.partial(jax.jit, static_argnames=())
def _sc_copy(src, zeros):
    mesh = plsc.VectorSubcoreMesh(core_axis_name="c", subcore_axis_name="s")
    return pl.kernel(
        _sc_body,
        out_type=jax.ShapeDtypeStruct((NUM_WINDOWS, WIN_PAD, D), jnp.float32),
        mesh=mesh,
        scratch_types=[
            pltpu.VMEM((IDXN,), jnp.int32),
            pltpu.VMEM((IDXN, D), jnp.float32),
            pltpu.SemaphoreType.DMA,
        ],
    )(src, zeros)


def kernel(audio_proj, num_frames):
    del num_frames
    src = audio_proj.reshape(SEQ_LEN, D)
    zeros = jnp.zeros((104, D), jnp.float32)
    padded = _sc_copy(src, zeros)
    sub_sequences = jnp.expand_dims(padded[:, :WIN, :], 0)
    k_lens = jnp.asarray(_K_LENS, dtype=jnp.int32)
    return sub_sequences, k_lens
